# speedup vs baseline: 183.3019x; 183.3019x over previous
import jax
import jax.numpy as jnp
from jax.experimental import pallas as pl
from jax.experimental.pallas import tpu as pltpu

_PATCH = 41
_PAD = _PATCH // 2
_OFFSETS = (1, 2, 4, 8, 16, 9)


def _dcp_body(rgb_ref, dcp_ref, sum_ref):
    neg_inf = jnp.float32(float("-inf"))
    r = rgb_ref[0, 0]
    g = rgb_ref[0, 1]
    b = rgb_ref[0, 2]
    m = jnp.maximum(jnp.maximum(-r, -g), -b)

    h, w = m.shape
    pad_rows = jnp.full((_PAD, w), neg_inf, jnp.float32)
    x = jnp.concatenate([pad_rows, m, pad_rows], axis=0)
    for s in _OFFSETS:
        x = jnp.maximum(x[:-s, :], x[s:, :])

    pad_cols = jnp.full((h, _PAD), neg_inf, jnp.float32)
    x = jnp.concatenate([pad_cols, x, pad_cols], axis=1)
    for s in _OFFSETS:
        x = jnp.maximum(x[:, :-s], x[:, s:])

    d = jnp.abs(x)
    dcp_ref[0, 0] = d
    sum_ref[0] = jnp.sum(d, keepdims=True)


def kernel(rgb):
    bsz, nch, h, w = rgb.shape
    dcp, sums = pl.pallas_call(
        _dcp_body,
        grid=(bsz,),
        in_specs=[pl.BlockSpec((1, nch, h, w), lambda i: (i, 0, 0, 0))],
        out_specs=[
            pl.BlockSpec((1, 1, h, w), lambda i: (i, 0, 0, 0)),
            pl.BlockSpec((1, 1, 1), lambda i: (i, 0, 0)),
        ],
        out_shape=[
            jax.ShapeDtypeStruct((bsz, 1, h, w), jnp.float32),
            jax.ShapeDtypeStruct((bsz, 1, 1), jnp.float32),
        ],
        compiler_params=pltpu.CompilerParams(
            dimension_semantics=("parallel",),
        ),
        name="dark_channel_prior",
    )(rgb)
    loss = jnp.sum(sums) / (bsz * h * w)
    return (loss, dcp)


# speedup vs baseline: 288.7022x; 1.5750x over previous
import jax
import jax.numpy as jnp
from jax.experimental import pallas as pl
from jax.experimental.pallas import tpu as pltpu

_PATCH = 41
_PAD = _PATCH // 2


def _slide_min_rows(m):
    pos_inf = jnp.float32(float("inf"))
    h, w = m.shape
    pad_rows = jnp.full((_PAD, w), pos_inf, jnp.float32)
    p = jnp.concatenate([pad_rows, m, pad_rows], axis=0)
    b2 = jnp.minimum(p[:-1, :], p[1:, :])
    b4 = jnp.minimum(b2[:-2, :], b2[2:, :])
    b8 = jnp.minimum(b4[:-4, :], b4[4:, :])
    b16 = jnp.minimum(b8[:-8, :], b8[8:, :])
    b32 = jnp.minimum(b16[:-16, :], b16[16:, :])
    b40 = jnp.minimum(b32[: h + 1, :], b8[32 : h + 33, :])
    return jnp.minimum(b40[:h, :], p[40 : h + 40, :])


def _dcp_body(rgb_ref, dcp_ref, sum_ref):
    r = rgb_ref[0, 0]
    g = rgb_ref[0, 1]
    b = rgb_ref[0, 2]
    m = jnp.minimum(jnp.minimum(r, g), b)

    x = _slide_min_rows(m)
    xt = jnp.transpose(x, (1, 0))
    qt = _slide_min_rows(xt)
    d = jnp.abs(jnp.transpose(qt, (1, 0)))
    dcp_ref[0, 0] = d
    sum_ref[0] = jnp.sum(d, keepdims=True)


def kernel(rgb):
    bsz, nch, h, w = rgb.shape
    dcp, sums = pl.pallas_call(
        _dcp_body,
        grid=(bsz,),
        in_specs=[pl.BlockSpec((1, nch, h, w), lambda i: (i, 0, 0, 0))],
        out_specs=[
            pl.BlockSpec((1, 1, h, w), lambda i: (i, 0, 0, 0)),
            pl.BlockSpec((1, 1, 1), lambda i: (i, 0, 0)),
        ],
        out_shape=[
            jax.ShapeDtypeStruct((bsz, 1, h, w), jnp.float32),
            jax.ShapeDtypeStruct((bsz, 1, 1), jnp.float32),
        ],
        compiler_params=pltpu.CompilerParams(
            dimension_semantics=("parallel",),
            vmem_limit_bytes=56 * 1024 * 1024,
        ),
        name="dark_channel_prior",
    )(rgb)
    loss = jnp.sum(sums) / (bsz * h * w)
    return (loss, dcp)


# speedup vs baseline: 332.2874x; 1.1510x over previous
import jax
import jax.numpy as jnp
from jax.experimental import pallas as pl
from jax.experimental.pallas import tpu as pltpu

_PATCH = 41
_PAD = _PATCH // 2


def _slide_min_rows(m):
    pos_inf = jnp.float32(float("inf"))
    h, w = m.shape
    pad_rows = jnp.full((_PAD, w), pos_inf, jnp.float32)
    p = jnp.concatenate([pad_rows, m, pad_rows], axis=0)
    b2 = jnp.minimum(p[:-1, :], p[1:, :])
    b4 = jnp.minimum(b2[:-2, :], b2[2:, :])
    b8 = jnp.minimum(b4[:-4, :], b4[4:, :])
    b16 = jnp.minimum(b8[:-8, :], b8[8:, :])
    b32 = jnp.minimum(b16[:-16, :], b16[16:, :])
    b40 = jnp.minimum(b32[: h + 1, :], b8[32 : h + 33, :])
    return jnp.minimum(b40[:h, :], p[40 : h + 40, :])


def _dcp_one(rgb_ref, dcp_ref, g):
    r = rgb_ref[g, 0]
    gg = rgb_ref[g, 1]
    b = rgb_ref[g, 2]
    m = jnp.minimum(jnp.minimum(r, gg), b)

    x = _slide_min_rows(m)
    xt = jnp.transpose(x, (1, 0))
    qt = _slide_min_rows(xt)
    d = jnp.abs(jnp.transpose(qt, (1, 0)))
    dcp_ref[g, 0] = d
    return jnp.sum(d, keepdims=True)


def _dcp_body(rgb_ref, dcp_ref, sum_ref):
    for g in range(rgb_ref.shape[0]):
        sum_ref[g] = _dcp_one(rgb_ref, dcp_ref, g)


def kernel(rgb):
    bsz, nch, h, w = rgb.shape
    gsz = 2
    dcp, sums = pl.pallas_call(
        _dcp_body,
        grid=(bsz // gsz,),
        in_specs=[pl.BlockSpec((gsz, nch, h, w), lambda i: (i, 0, 0, 0))],
        out_specs=[
            pl.BlockSpec((gsz, 1, h, w), lambda i: (i, 0, 0, 0)),
            pl.BlockSpec((gsz, 1, 1), lambda i: (i, 0, 0)),
        ],
        out_shape=[
            jax.ShapeDtypeStruct((bsz, 1, h, w), jnp.float32),
            jax.ShapeDtypeStruct((bsz, 1, 1), jnp.float32),
        ],

        compiler_params=pltpu.CompilerParams(
            dimension_semantics=("parallel",),
            vmem_limit_bytes=56 * 1024 * 1024,
        ),
        name="dark_channel_prior",
    )(rgb)
    loss = jnp.sum(sums) / (bsz * h * w)
    return (loss, dcp)


# speedup vs baseline: 341.0291x; 1.0263x over previous
import jax
import jax.numpy as jnp
from jax.experimental import pallas as pl
from jax.experimental.pallas import tpu as pltpu

_PATCH = 41
_PAD = _PATCH // 2


def _slide_min_rows(m):
    pos_inf = jnp.float32(float("inf"))
    h, w = m.shape
    pad_rows = jnp.full((_PAD, w), pos_inf, jnp.float32)
    p = jnp.concatenate([pad_rows, m, pad_rows], axis=0)
    b2 = jnp.minimum(p[:-1, :], p[1:, :])
    b4 = jnp.minimum(b2[:-2, :], b2[2:, :])
    b8 = jnp.minimum(b4[:-4, :], b4[4:, :])
    b16 = jnp.minimum(b8[:-8, :], b8[8:, :])
    b32 = jnp.minimum(b16[:-16, :], b16[16:, :])
    b40 = jnp.minimum(b32[: h + 1, :], b8[32 : h + 33, :])
    return jnp.minimum(b40[:h, :], p[40 : h + 40, :])


def _dcp_one(rgb_ref, dcp_ref, g):
    r = rgb_ref[g, 0]
    gg = rgb_ref[g, 1]
    b = rgb_ref[g, 2]
    m = jnp.minimum(jnp.minimum(r, gg), b)

    x = _slide_min_rows(m)
    xt = jnp.transpose(x, (1, 0))
    qt = _slide_min_rows(xt)
    d = jnp.abs(jnp.transpose(qt, (1, 0)))
    dcp_ref[g, 0] = d
    return jnp.sum(d, keepdims=True)


def _dcp_body(rgb_ref, dcp_ref, sum_ref):
    @pl.when(pl.program_id(0) == 0)
    def _():
        sum_ref[...] = jnp.zeros_like(sum_ref)

    part = jnp.zeros((1, 1), jnp.float32)
    for g in range(rgb_ref.shape[0]):
        part = part + _dcp_one(rgb_ref, dcp_ref, g)
    sum_ref[...] += part


def kernel(rgb):
    bsz, nch, h, w = rgb.shape
    gsz = 2
    dcp, sums = pl.pallas_call(
        _dcp_body,
        grid=(bsz // gsz,),
        in_specs=[pl.BlockSpec((gsz, nch, h, w), lambda i: (i, 0, 0, 0))],
        out_specs=[
            pl.BlockSpec((gsz, 1, h, w), lambda i: (i, 0, 0, 0)),
            pl.BlockSpec((1, 1), lambda i: (0, 0)),
        ],
        out_shape=[
            jax.ShapeDtypeStruct((bsz, 1, h, w), jnp.float32),
            jax.ShapeDtypeStruct((1, 1), jnp.float32),
        ],
        compiler_params=pltpu.CompilerParams(
            dimension_semantics=("arbitrary",),
            vmem_limit_bytes=56 * 1024 * 1024,
        ),
        name="dark_channel_prior",
    )(rgb)
    loss = sums[0, 0] / (bsz * h * w)
    return (loss, dcp)


# speedup vs baseline: 356.5966x; 1.0456x over previous
import jax
import jax.numpy as jnp
from jax.experimental import pallas as pl
from jax.experimental.pallas import tpu as pltpu

_PATCH = 41
_PAD = _PATCH // 2


def _slide_min_rows(m):
    pos_inf = jnp.float32(float("inf"))
    h, w = m.shape
    pad_rows = jnp.full((_PAD, w), pos_inf, jnp.float32)
    p = jnp.concatenate([pad_rows, m, pad_rows], axis=0)
    b2 = jnp.minimum(p[:-1, :], p[1:, :])
    b4 = jnp.minimum(b2[:-2, :], b2[2:, :])
    b8 = jnp.minimum(b4[:-4, :], b4[4:, :])
    b16 = jnp.minimum(b8[:-8, :], b8[8:, :])
    b32 = jnp.minimum(b16[:-16, :], b16[16:, :])
    b40 = jnp.minimum(b32[: h + 1, :], b8[32 : h + 33, :])
    return jnp.minimum(b40[:h, :], p[40 : h + 40, :])


def _dcp_one(rgb_ref, dcp_ref, g):
    r = rgb_ref[g, 0]
    gg = rgb_ref[g, 1]
    b = rgb_ref[g, 2]
    m = jnp.minimum(jnp.minimum(r, gg), b)

    x = _slide_min_rows(m)
    xt = jnp.transpose(x, (1, 0))
    qt = _slide_min_rows(xt)
    d = jnp.abs(jnp.transpose(qt, (1, 0)))
    dcp_ref[g, 0] = d
    return jnp.sum(d, keepdims=True)


def _dcp_body(rgb_ref, dcp_ref, sum_ref):
    @pl.when(pl.program_id(0) == 0)
    def _():
        sum_ref[...] = jnp.zeros_like(sum_ref)

    part = jnp.zeros((1, 1), jnp.float32)
    for g in range(rgb_ref.shape[0]):
        part = part + _dcp_one(rgb_ref, dcp_ref, g)
    sum_ref[...] += part


def kernel(rgb):
    bsz, nch, h, w = rgb.shape
    gsz = 4
    dcp, sums = pl.pallas_call(
        _dcp_body,
        grid=(bsz // gsz,),
        in_specs=[pl.BlockSpec((gsz, nch, h, w), lambda i: (i, 0, 0, 0))],
        out_specs=[
            pl.BlockSpec((gsz, 1, h, w), lambda i: (i, 0, 0, 0)),
            pl.BlockSpec((1, 1), lambda i: (0, 0)),
        ],
        out_shape=[
            jax.ShapeDtypeStruct((bsz, 1, h, w), jnp.float32),
            jax.ShapeDtypeStruct((1, 1), jnp.float32),
        ],
        compiler_params=pltpu.CompilerParams(
            dimension_semantics=("arbitrary",),
            vmem_limit_bytes=56 * 1024 * 1024,
        ),
        name="dark_channel_prior",
    )(rgb)
    loss = sums[0, 0] / (bsz * h * w)
    return (loss, dcp)
